# Initial kernel scaffold; baseline (speedup 1.0000x reference)
import jax
import jax.numpy as jnp
from jax.experimental import pallas as pl


def kernel(unknown, known, unknow_feats, known_feats, W0, g0, be0, W1, g1, be1):
    raise NotImplementedError("write your pallas kernel here")


# speedup vs baseline: 8.9848x; 8.9848x over previous
import functools

import jax
import jax.numpy as jnp
from jax import lax
from jax.experimental import pallas as pl
from jax.experimental.pallas import tpu as pltpu
from jax.experimental.pallas import tpu_sc as plsc

B, N, M = 8, 4096, 1024
C1, C2 = 256, 512
CO = 256

TN = 512
TN2 = 512
NT = N // TN
NT2 = N // TN2

NW = 32
PPW = (B * N) // NW
CH = 32
K = 3


def _knn_body(u_ref, k_ref, gidx_ref, w0_ref, w1_ref, w2_ref):
    b = pl.program_id(0)
    u = u_ref[0]
    kp = k_ref[0]
    uk = lax.dot_general(u, kp, (((1,), (1,)), ((), ())),
                         preferred_element_type=jnp.float32)
    u2 = jnp.sum(u * u, axis=1, keepdims=True)
    k2 = jnp.sum(kp * kp, axis=1)[None, :]
    d = jnp.maximum(u2 + k2 - 2.0 * uk, 0.0)
    iota = lax.broadcasted_iota(jnp.int32, d.shape, 1)
    idxs, recips = [], []
    for _ in range(K):
        mk = jnp.min(d, axis=1, keepdims=True)
        am = jnp.min(jnp.where(d == mk, iota, M), axis=1, keepdims=True)
        idxs.append(am)
        recips.append(1.0 / (jnp.sqrt(mk) + 1e-8))
        d = jnp.where(iota == am, jnp.float32(jnp.inf), d)
    idx3 = jnp.concatenate(idxs, axis=1)
    norm = recips[0] + recips[1] + recips[2]
    gidx_ref[...] = idx3 + b * M
    for r, ref in zip(recips, (w0_ref, w1_ref, w2_ref)):
        ref[...] = jnp.broadcast_to(r / norm, (TN, 16))


def _knn(unknown, known):
    return pl.pallas_call(
        _knn_body,
        grid=(B, NT),
        in_specs=[
            pl.BlockSpec((1, TN, 3), lambda b, t: (b, t, 0)),
            pl.BlockSpec((1, M, 3), lambda b, t: (b, 0, 0)),
        ],
        out_specs=[
            pl.BlockSpec((TN, K), lambda b, t: (b * NT + t, 0)),
            pl.BlockSpec((TN, 16), lambda b, t: (b * NT + t, 0)),
            pl.BlockSpec((TN, 16), lambda b, t: (b * NT + t, 0)),
            pl.BlockSpec((TN, 16), lambda b, t: (b * NT + t, 0)),
        ],
        out_shape=[
            jax.ShapeDtypeStruct((B * N, K), jnp.int32),
            jax.ShapeDtypeStruct((B * N, 16), jnp.float32),
            jax.ShapeDtypeStruct((B * N, 16), jnp.float32),
            jax.ShapeDtypeStruct((B * N, 16), jnp.float32),
        ],
    )(unknown, known)


def _sc_interp_body(gidx_hbm, w0_hbm, w1_hbm, w2_hbm, kft_hbm, out_hbm,
                    idx_v, w0_v, w1_v, w2_v, rows_v, out_v, sem):
    wid = lax.axis_index("s") * 2 + lax.axis_index("c")

    def chunk(ci, carry):
        base = wid * PPW + ci * CH
        pltpu.sync_copy(gidx_hbm.at[pl.ds(base * K, CH * K)], idx_v)
        pltpu.sync_copy(w0_hbm.at[pl.ds(base, CH)], w0_v)
        pltpu.sync_copy(w1_hbm.at[pl.ds(base, CH)], w1_v)
        pltpu.sync_copy(w2_hbm.at[pl.ds(base, CH)], w2_v)
        pltpu.async_copy(kft_hbm.at[idx_v], rows_v, sem).wait()

        def point(p, c2):
            i0 = 3 * p
            wa = w0_v[p, :]
            wb = w1_v[p, :]
            wc = w2_v[p, :]
            for c in range(C2 // 16):
                sl = pl.ds(c * 16, 16)
                acc = wa * rows_v[i0, sl]
                acc = acc + wb * rows_v[i0 + 1, sl]
                acc = acc + wc * rows_v[i0 + 2, sl]
                out_v[p, sl] = acc
            return c2

        lax.fori_loop(0, CH, point, 0)
        pltpu.sync_copy(out_v, out_hbm.at[pl.ds(base, CH)])
        return carry

    lax.fori_loop(0, PPW // CH, chunk, 0)


def _sc_interp(gidx_flat, w0e, w1e, w2e, kft):
    mesh = plsc.VectorSubcoreMesh(core_axis_name="c", subcore_axis_name="s")
    f = functools.partial(
        pl.kernel,
        out_type=jax.ShapeDtypeStruct((B * N, C2), jnp.float32),
        mesh=mesh,
        scratch_types=[
            pltpu.VMEM((CH * K,), jnp.int32),
            pltpu.VMEM((CH, 16), jnp.float32),
            pltpu.VMEM((CH, 16), jnp.float32),
            pltpu.VMEM((CH, 16), jnp.float32),
            pltpu.VMEM((CH * K, C2), jnp.float32),
            pltpu.VMEM((CH, C2), jnp.float32),
            pltpu.SemaphoreType.DMA,
        ],
    )(_sc_interp_body)
    return f(gidx_flat, w0e, w1e, w2e, kft)


def _mlp1_body(it_ref, uf_ref, w0a_ref, w0b_ref, y1_ref, s1_ref):
    step = pl.program_id(0) * pl.num_programs(1) + pl.program_id(1)
    it = it_ref[...]
    uf = uf_ref[0]
    y = lax.dot_general(w0a_ref[...], it, (((1,), (1,)), ((), ())),
                        preferred_element_type=jnp.float32)
    y = y + lax.dot_general(w0b_ref[...], uf, (((1,), (0,)), ((), ())),
                            preferred_element_type=jnp.float32)
    y1_ref[0] = y
    st = jnp.concatenate([jnp.sum(y, axis=1)[None, :],
                          jnp.sum(y * y, axis=1)[None, :]], axis=0)

    @pl.when(step == 0)
    def _():
        s1_ref[...] = st

    @pl.when(step != 0)
    def _():
        s1_ref[...] += st


def _mlp1(interp, uf, w0a, w0b):
    return pl.pallas_call(
        _mlp1_body,
        grid=(B, NT2),
        in_specs=[
            pl.BlockSpec((TN2, C2), lambda b, t: (b * NT2 + t, 0)),
            pl.BlockSpec((1, C1, TN2), lambda b, t: (b, 0, t)),
            pl.BlockSpec((CO, C2), lambda b, t: (0, 0)),
            pl.BlockSpec((CO, C1), lambda b, t: (0, 0)),
        ],
        out_specs=[
            pl.BlockSpec((1, CO, TN2), lambda b, t: (b, 0, t)),
            pl.BlockSpec((2, CO), lambda b, t: (0, 0)),
        ],
        out_shape=[
            jax.ShapeDtypeStruct((B, CO, N), jnp.float32),
            jax.ShapeDtypeStruct((2, CO), jnp.float32),
        ],
    )(interp, uf, w0a, w0b)


def _mlp2_body(y1_ref, s1_ref, g_ref, be_ref, w1_ref, y2_ref, s2_ref):
    step = pl.program_id(0) * pl.num_programs(1) + pl.program_id(1)
    n = jnp.float32(B * N)
    mean = s1_ref[0, :] / n
    var = s1_ref[1, :] / n - mean * mean
    scale = g_ref[0] / jnp.sqrt(var + 1e-5)
    shift = be_ref[0] - mean * scale
    h = jnp.maximum(y1_ref[0] * scale[:, None] + shift[:, None], 0.0)
    y = lax.dot_general(w1_ref[...], h, (((1,), (0,)), ((), ())),
                        preferred_element_type=jnp.float32)
    y2_ref[0] = y
    st = jnp.concatenate([jnp.sum(y, axis=1)[None, :],
                          jnp.sum(y * y, axis=1)[None, :]], axis=0)

    @pl.when(step == 0)
    def _():
        s2_ref[...] = st

    @pl.when(step != 0)
    def _():
        s2_ref[...] += st


def _mlp2(y1, s1, g0, be0, w1):
    return pl.pallas_call(
        _mlp2_body,
        grid=(B, NT2),
        in_specs=[
            pl.BlockSpec((1, CO, TN2), lambda b, t: (b, 0, t)),
            pl.BlockSpec((2, CO), lambda b, t: (0, 0)),
            pl.BlockSpec((1, CO), lambda b, t: (0, 0)),
            pl.BlockSpec((1, CO), lambda b, t: (0, 0)),
            pl.BlockSpec((CO, CO), lambda b, t: (0, 0)),
        ],
        out_specs=[
            pl.BlockSpec((1, CO, TN2), lambda b, t: (b, 0, t)),
            pl.BlockSpec((2, CO), lambda b, t: (0, 0)),
        ],
        out_shape=[
            jax.ShapeDtypeStruct((B, CO, N), jnp.float32),
            jax.ShapeDtypeStruct((2, CO), jnp.float32),
        ],
    )(y1, s1, g0, be0, w1)


def _bnout_body(y2_ref, s2_ref, g_ref, be_ref, out_ref):
    n = jnp.float32(B * N)
    mean = s2_ref[0, :] / n
    var = s2_ref[1, :] / n - mean * mean
    scale = g_ref[0] / jnp.sqrt(var + 1e-5)
    shift = be_ref[0] - mean * scale
    out_ref[0] = jnp.maximum(y2_ref[0] * scale[:, None] + shift[:, None], 0.0)


def _bnout(y2, s2, g1, be1):
    return pl.pallas_call(
        _bnout_body,
        grid=(B, NT2),
        in_specs=[
            pl.BlockSpec((1, CO, TN2), lambda b, t: (b, 0, t)),
            pl.BlockSpec((2, CO), lambda b, t: (0, 0)),
            pl.BlockSpec((1, CO), lambda b, t: (0, 0)),
            pl.BlockSpec((1, CO), lambda b, t: (0, 0)),
        ],
        out_specs=pl.BlockSpec((1, CO, TN2), lambda b, t: (b, 0, t)),
        out_shape=jax.ShapeDtypeStruct((B, CO, N), jnp.float32),
    )(y2, s2, g1, be1)


def kernel(unknown, known, unknow_feats, known_feats, W0, g0, be0, W1, g1, be1):
    gidx, w0e, w1e, w2e = _knn(unknown, known)
    kft = jnp.transpose(known_feats, (0, 2, 1)).reshape(B * M, C2)
    interp = _sc_interp(gidx.reshape(B * N * K), w0e, w1e, w2e, kft)
    w0a = W0[:, :C2]
    w0b = W0[:, C2:]
    y1, s1 = _mlp1(interp, unknow_feats, w0a, w0b)
    y2, s2 = _mlp2(y1, s1, g0.reshape(1, CO), be0.reshape(1, CO), W1)
    return _bnout(y2, s2, g1.reshape(1, CO), be1.reshape(1, CO))


# speedup vs baseline: 11.3329x; 1.2613x over previous
import functools

import jax
import jax.numpy as jnp
from jax import lax
from jax.experimental import pallas as pl
from jax.experimental.pallas import tpu as pltpu
from jax.experimental.pallas import tpu_sc as plsc

B, N, M = 8, 4096, 1024
C1, C2 = 256, 512
CO = 256

TN = 512
TN2 = 512
NT = N // TN
NT2 = N // TN2

NW = 32
PPW = (B * N) // NW
CH = 16
NCH = PPW // CH
K = 3


def _knn_body(u_ref, k_ref, gidx_ref, w_ref):
    b = pl.program_id(0)
    u = u_ref[0]
    kp = k_ref[0]
    uk = lax.dot_general(u, kp, (((1,), (1,)), ((), ())),
                         preferred_element_type=jnp.float32)
    u2 = jnp.sum(u * u, axis=1, keepdims=True)
    k2 = jnp.sum(kp * kp, axis=1)[None, :]
    d = jnp.maximum(u2 + k2 - 2.0 * uk, 0.0)
    iota = lax.broadcasted_iota(jnp.int32, d.shape, 1)
    idxs, recips = [], []
    for _ in range(K):
        mk = jnp.min(d, axis=1, keepdims=True)
        am = jnp.min(jnp.where(d == mk, iota, M), axis=1, keepdims=True)
        idxs.append(am)
        recips.append(1.0 / (jnp.sqrt(mk) + 1e-8))
        d = jnp.where(iota == am, jnp.float32(jnp.inf), d)
    idx3 = jnp.concatenate(idxs, axis=1)
    norm = recips[0] + recips[1] + recips[2]
    gidx_ref[...] = idx3 + b * M
    w_ref[...] = jnp.concatenate(
        [jnp.broadcast_to(r / norm, (TN, 16)) for r in recips], axis=1)


def _knn(unknown, known):
    return pl.pallas_call(
        _knn_body,
        grid=(B, NT),
        in_specs=[
            pl.BlockSpec((1, TN, 3), lambda b, t: (b, t, 0)),
            pl.BlockSpec((1, M, 3), lambda b, t: (b, 0, 0)),
        ],
        out_specs=[
            pl.BlockSpec((TN, K), lambda b, t: (b * NT + t, 0)),
            pl.BlockSpec((TN, 48), lambda b, t: (b * NT + t, 0)),
        ],
        out_shape=[
            jax.ShapeDtypeStruct((B * N, K), jnp.int32),
            jax.ShapeDtypeStruct((B * N, 48), jnp.float32),
        ],
    )(unknown, known)


def _sc_interp_body(gidx_hbm, w_hbm, kft_hbm, out_hbm,
                    idx_v, w_v, rows0, rows1, out0, out1,
                    gsem0, gsem1, osem0, osem1):
    wid = lax.axis_index("s") * 2 + lax.axis_index("c")
    base = wid * PPW
    pltpu.sync_copy(gidx_hbm.at[pl.ds(base * K, PPW * K)], idx_v)
    pltpu.sync_copy(w_hbm.at[pl.ds(base * 48, PPW * 48)], w_v)
    pltpu.async_copy(kft_hbm.at[idx_v.at[pl.ds(0, CH * K)]], rows0, gsem0)

    def _compute(ci, rows_v, out_v):
        p0 = ci * CH

        def point(j, c2):
            gp = p0 + j
            i0 = K * j
            wo = gp * 48
            wa = w_v[pl.ds(wo, 16)]
            wb = w_v[pl.ds(wo + 16, 16)]
            wc = w_v[pl.ds(wo + 32, 16)]
            for c in range(C2 // 16):
                sl = pl.ds(c * 16, 16)
                acc = wa * rows_v[i0, sl]
                acc = acc + wb * rows_v[i0 + 1, sl]
                acc = acc + wc * rows_v[i0 + 2, sl]
                out_v[j, sl] = acc
            return c2

        lax.fori_loop(0, CH, point, 0)

    def _wait_gather(rows_v, sem):
        pltpu.make_async_copy(kft_hbm.at[pl.ds(0, CH * K)], rows_v, sem).wait()

    def _wait_store(out_v, sem):
        pltpu.make_async_copy(out_hbm.at[pl.ds(0, CH)], out_v, sem).wait()

    def pair(i, carry):
        c0 = 2 * i
        c1 = 2 * i + 1
        pltpu.async_copy(kft_hbm.at[idx_v.at[pl.ds(c1 * CH * K, CH * K)]],
                         rows1, gsem1)
        _wait_gather(rows0, gsem0)

        @pl.when(i > 0)
        def _():
            _wait_store(out0, osem0)

        _compute(c0, rows0, out0)
        pltpu.async_copy(out0, out_hbm.at[pl.ds(base + c0 * CH, CH)], osem0)

        @pl.when(i < NCH // 2 - 1)
        def _():
            pltpu.async_copy(
                kft_hbm.at[idx_v.at[pl.ds((c0 + 2) * CH * K, CH * K)]],
                rows0, gsem0)

        _wait_gather(rows1, gsem1)

        @pl.when(i > 0)
        def _():
            _wait_store(out1, osem1)

        _compute(c1, rows1, out1)
        pltpu.async_copy(out1, out_hbm.at[pl.ds(base + c1 * CH, CH)], osem1)
        return carry

    lax.fori_loop(0, NCH // 2, pair, 0)
    _wait_store(out0, osem0)
    _wait_store(out1, osem1)


def _sc_interp(gidx_flat, w48, kft):
    mesh = plsc.VectorSubcoreMesh(core_axis_name="c", subcore_axis_name="s")
    f = functools.partial(
        pl.kernel,
        out_type=jax.ShapeDtypeStruct((B * N, C2), jnp.float32),
        mesh=mesh,
        scratch_types=[
            pltpu.VMEM((PPW * K,), jnp.int32),
            pltpu.VMEM((PPW * 48,), jnp.float32),
            pltpu.VMEM((CH * K, C2), jnp.float32),
            pltpu.VMEM((CH * K, C2), jnp.float32),
            pltpu.VMEM((CH, C2), jnp.float32),
            pltpu.VMEM((CH, C2), jnp.float32),
            pltpu.SemaphoreType.DMA,
            pltpu.SemaphoreType.DMA,
            pltpu.SemaphoreType.DMA,
            pltpu.SemaphoreType.DMA,
        ],
    )(_sc_interp_body)
    return f(gidx_flat, w48, kft)


def _mlp1_body(it_ref, uf_ref, w0a_ref, w0b_ref, y1_ref, s1_ref):
    step = pl.program_id(0) * pl.num_programs(1) + pl.program_id(1)
    it = it_ref[...]
    uf = uf_ref[0]
    y = lax.dot_general(w0a_ref[...], it, (((1,), (1,)), ((), ())),
                        preferred_element_type=jnp.float32)
    y = y + lax.dot_general(w0b_ref[...], uf, (((1,), (0,)), ((), ())),
                            preferred_element_type=jnp.float32)
    y1_ref[0] = y
    st = jnp.concatenate([jnp.sum(y, axis=1)[None, :],
                          jnp.sum(y * y, axis=1)[None, :]], axis=0)

    @pl.when(step == 0)
    def _():
        s1_ref[...] = st

    @pl.when(step != 0)
    def _():
        s1_ref[...] += st


def _mlp1(interp, uf, w0a, w0b):
    return pl.pallas_call(
        _mlp1_body,
        grid=(B, NT2),
        in_specs=[
            pl.BlockSpec((TN2, C2), lambda b, t: (b * NT2 + t, 0)),
            pl.BlockSpec((1, C1, TN2), lambda b, t: (b, 0, t)),
            pl.BlockSpec((CO, C2), lambda b, t: (0, 0)),
            pl.BlockSpec((CO, C1), lambda b, t: (0, 0)),
        ],
        out_specs=[
            pl.BlockSpec((1, CO, TN2), lambda b, t: (b, 0, t)),
            pl.BlockSpec((2, CO), lambda b, t: (0, 0)),
        ],
        out_shape=[
            jax.ShapeDtypeStruct((B, CO, N), jnp.float32),
            jax.ShapeDtypeStruct((2, CO), jnp.float32),
        ],
    )(interp, uf, w0a, w0b)


def _mlp2_body(y1_ref, s1_ref, g_ref, be_ref, w1_ref, y2_ref, s2_ref):
    step = pl.program_id(0) * pl.num_programs(1) + pl.program_id(1)
    n = jnp.float32(B * N)
    mean = s1_ref[0, :] / n
    var = s1_ref[1, :] / n - mean * mean
    scale = g_ref[0] / jnp.sqrt(var + 1e-5)
    shift = be_ref[0] - mean * scale
    h = jnp.maximum(y1_ref[0] * scale[:, None] + shift[:, None], 0.0)
    y = lax.dot_general(w1_ref[...], h, (((1,), (0,)), ((), ())),
                        preferred_element_type=jnp.float32)
    y2_ref[0] = y
    st = jnp.concatenate([jnp.sum(y, axis=1)[None, :],
                          jnp.sum(y * y, axis=1)[None, :]], axis=0)

    @pl.when(step == 0)
    def _():
        s2_ref[...] = st

    @pl.when(step != 0)
    def _():
        s2_ref[...] += st


def _mlp2(y1, s1, g0, be0, w1):
    return pl.pallas_call(
        _mlp2_body,
        grid=(B, NT2),
        in_specs=[
            pl.BlockSpec((1, CO, TN2), lambda b, t: (b, 0, t)),
            pl.BlockSpec((2, CO), lambda b, t: (0, 0)),
            pl.BlockSpec((1, CO), lambda b, t: (0, 0)),
            pl.BlockSpec((1, CO), lambda b, t: (0, 0)),
            pl.BlockSpec((CO, CO), lambda b, t: (0, 0)),
        ],
        out_specs=[
            pl.BlockSpec((1, CO, TN2), lambda b, t: (b, 0, t)),
            pl.BlockSpec((2, CO), lambda b, t: (0, 0)),
        ],
        out_shape=[
            jax.ShapeDtypeStruct((B, CO, N), jnp.float32),
            jax.ShapeDtypeStruct((2, CO), jnp.float32),
        ],
    )(y1, s1, g0, be0, w1)


def _bnout_body(y2_ref, s2_ref, g_ref, be_ref, out_ref):
    n = jnp.float32(B * N)
    mean = s2_ref[0, :] / n
    var = s2_ref[1, :] / n - mean * mean
    scale = g_ref[0] / jnp.sqrt(var + 1e-5)
    shift = be_ref[0] - mean * scale
    out_ref[0] = jnp.maximum(y2_ref[0] * scale[:, None] + shift[:, None], 0.0)


def _bnout(y2, s2, g1, be1):
    return pl.pallas_call(
        _bnout_body,
        grid=(B, NT2),
        in_specs=[
            pl.BlockSpec((1, CO, TN2), lambda b, t: (b, 0, t)),
            pl.BlockSpec((2, CO), lambda b, t: (0, 0)),
            pl.BlockSpec((1, CO), lambda b, t: (0, 0)),
            pl.BlockSpec((1, CO), lambda b, t: (0, 0)),
        ],
        out_specs=pl.BlockSpec((1, CO, TN2), lambda b, t: (b, 0, t)),
        out_shape=jax.ShapeDtypeStruct((B, CO, N), jnp.float32),
    )(y2, s2, g1, be1)


def kernel(unknown, known, unknow_feats, known_feats, W0, g0, be0, W1, g1, be1):
    gidx, w48 = _knn(unknown, known)
    kft = jnp.transpose(known_feats, (0, 2, 1)).reshape(B * M, C2)
    interp = _sc_interp(gidx.reshape(B * N * K), w48.reshape(B * N * 48), kft)
    w0a = W0[:, :C2]
    w0b = W0[:, C2:]
    y1, s1 = _mlp1(interp, unknow_feats, w0a, w0b)
    y2, s2 = _mlp2(y1, s1, g0.reshape(1, CO), be0.reshape(1, CO), W1)
    return _bnout(y2, s2, g1.reshape(1, CO), be1.reshape(1, CO))


# speedup vs baseline: 14.9273x; 1.3172x over previous
import functools

import jax
import jax.numpy as jnp
from jax import lax
from jax.experimental import pallas as pl
from jax.experimental.pallas import tpu as pltpu
from jax.experimental.pallas import tpu_sc as plsc

B, N, M = 8, 4096, 1024
C1, C2 = 256, 512
CO = 256

TN = 512
TN2 = 512
NT = N // TN
NT2 = N // TN2

NW = 32
PPW = (B * N) // NW
CH = 16
NCH = PPW // CH
K = 3


def _knn_body(u_ref, k_ref, gidx_ref, w_ref):
    b = pl.program_id(0)
    u = u_ref[0]
    kp = k_ref[0]
    uk = lax.dot_general(u, kp, (((1,), (1,)), ((), ())),
                         preferred_element_type=jnp.float32)
    u2 = jnp.sum(u * u, axis=1, keepdims=True)
    k2 = jnp.sum(kp * kp, axis=1)[None, :]
    d = jnp.maximum(u2 + k2 - 2.0 * uk, 0.0)
    iota = lax.broadcasted_iota(jnp.int32, d.shape, 1).astype(jnp.float32)
    idxs, recips = [], []
    for k in range(K):
        mk = jnp.min(d, axis=1, keepdims=True)
        am = jnp.min(jnp.where(d == mk, iota, jnp.float32(M)),
                     axis=1, keepdims=True)
        idxs.append(am)
        recips.append(1.0 / (jnp.sqrt(mk) + 1e-8))
        if k < K - 1:
            d = jnp.where(iota == am, jnp.float32(jnp.inf), d)
    idx3 = jnp.concatenate(idxs, axis=1).astype(jnp.int32)
    norm = recips[0] + recips[1] + recips[2]
    gidx_ref[...] = idx3 + b * M
    w_ref[...] = jnp.concatenate(
        [jnp.broadcast_to(r / norm, (TN, 16)) for r in recips], axis=1)


def _knn(unknown, known):
    return pl.pallas_call(
        _knn_body,
        grid=(B, NT),
        in_specs=[
            pl.BlockSpec((1, TN, 3), lambda b, t: (b, t, 0)),
            pl.BlockSpec((1, M, 3), lambda b, t: (b, 0, 0)),
        ],
        out_specs=[
            pl.BlockSpec((TN, K), lambda b, t: (b * NT + t, 0)),
            pl.BlockSpec((TN, 48), lambda b, t: (b * NT + t, 0)),
        ],
        out_shape=[
            jax.ShapeDtypeStruct((B * N, K), jnp.int32),
            jax.ShapeDtypeStruct((B * N, 48), jnp.float32),
        ],
    )(unknown, known)


def _sc_interp_body(gidx_hbm, w_hbm, kft_hbm, out_hbm,
                    idx_v, w_v, rows0, rows1, out0, out1,
                    gsem0, gsem1, osem0, osem1):
    wid = lax.axis_index("s") * 2 + lax.axis_index("c")
    base = wid * PPW
    pltpu.sync_copy(gidx_hbm.at[pl.ds(base * K, PPW * K)], idx_v)
    pltpu.sync_copy(w_hbm.at[pl.ds(base * 48, PPW * 48)], w_v)
    pltpu.async_copy(kft_hbm.at[idx_v.at[pl.ds(0, CH * K)]], rows0, gsem0)

    def _compute(ci, rows_v, out_v):
        p0 = ci * CH

        @plsc.parallel_loop(0, CH, 1)
        def point(j):
            gp = p0 + j
            i0 = K * j
            wo = gp * 48
            wa = w_v[pl.ds(wo, 16)]
            wb = w_v[pl.ds(wo + 16, 16)]
            wc = w_v[pl.ds(wo + 32, 16)]
            for c in range(C2 // 16):
                sl = pl.ds(c * 16, 16)
                acc = wa * rows_v[i0, sl]
                acc = acc + wb * rows_v[i0 + 1, sl]
                acc = acc + wc * rows_v[i0 + 2, sl]
                out_v[j, sl] = acc

    def _wait_gather(rows_v, sem):
        pltpu.make_async_copy(kft_hbm.at[pl.ds(0, CH * K)], rows_v, sem).wait()

    def _wait_store(out_v, sem):
        pltpu.make_async_copy(out_hbm.at[pl.ds(0, CH)], out_v, sem).wait()

    def pair(i, carry):
        c0 = 2 * i
        c1 = 2 * i + 1
        pltpu.async_copy(kft_hbm.at[idx_v.at[pl.ds(c1 * CH * K, CH * K)]],
                         rows1, gsem1)
        _wait_gather(rows0, gsem0)

        @pl.when(i > 0)
        def _():
            _wait_store(out0, osem0)

        _compute(c0, rows0, out0)
        pltpu.async_copy(out0, out_hbm.at[pl.ds(base + c0 * CH, CH)], osem0)

        @pl.when(i < NCH // 2 - 1)
        def _():
            pltpu.async_copy(
                kft_hbm.at[idx_v.at[pl.ds((c0 + 2) * CH * K, CH * K)]],
                rows0, gsem0)

        _wait_gather(rows1, gsem1)

        @pl.when(i > 0)
        def _():
            _wait_store(out1, osem1)

        _compute(c1, rows1, out1)
        pltpu.async_copy(out1, out_hbm.at[pl.ds(base + c1 * CH, CH)], osem1)
        return carry

    lax.fori_loop(0, NCH // 2, pair, 0)
    _wait_store(out0, osem0)
    _wait_store(out1, osem1)


def _sc_interp(gidx_flat, w48, kft):
    mesh = plsc.VectorSubcoreMesh(core_axis_name="c", subcore_axis_name="s")
    f = functools.partial(
        pl.kernel,
        out_type=jax.ShapeDtypeStruct((B * N, C2), jnp.float32),
        mesh=mesh,
        scratch_types=[
            pltpu.VMEM((PPW * K,), jnp.int32),
            pltpu.VMEM((PPW * 48,), jnp.float32),
            pltpu.VMEM((CH * K, C2), jnp.float32),
            pltpu.VMEM((CH * K, C2), jnp.float32),
            pltpu.VMEM((CH, C2), jnp.float32),
            pltpu.VMEM((CH, C2), jnp.float32),
            pltpu.SemaphoreType.DMA,
            pltpu.SemaphoreType.DMA,
            pltpu.SemaphoreType.DMA,
            pltpu.SemaphoreType.DMA,
        ],
    )(_sc_interp_body)
    return f(gidx_flat, w48, kft)


def _mlp1_body(it_ref, uf_ref, w0a_ref, w0b_ref, y1_ref, s1_ref):
    step = pl.program_id(0) * pl.num_programs(1) + pl.program_id(1)
    it = it_ref[...]
    uf = uf_ref[0]
    y = lax.dot_general(w0a_ref[...], it, (((1,), (1,)), ((), ())),
                        preferred_element_type=jnp.float32)
    y = y + lax.dot_general(w0b_ref[...], uf, (((1,), (0,)), ((), ())),
                            preferred_element_type=jnp.float32)
    y1_ref[0] = y
    st = jnp.concatenate([jnp.sum(y, axis=1)[None, :],
                          jnp.sum(y * y, axis=1)[None, :]], axis=0)

    @pl.when(step == 0)
    def _():
        s1_ref[...] = st

    @pl.when(step != 0)
    def _():
        s1_ref[...] += st


def _mlp1(interp, uf, w0a, w0b):
    return pl.pallas_call(
        _mlp1_body,
        grid=(B, NT2),
        in_specs=[
            pl.BlockSpec((TN2, C2), lambda b, t: (b * NT2 + t, 0)),
            pl.BlockSpec((1, C1, TN2), lambda b, t: (b, 0, t)),
            pl.BlockSpec((CO, C2), lambda b, t: (0, 0)),
            pl.BlockSpec((CO, C1), lambda b, t: (0, 0)),
        ],
        out_specs=[
            pl.BlockSpec((1, CO, TN2), lambda b, t: (b, 0, t)),
            pl.BlockSpec((2, CO), lambda b, t: (0, 0)),
        ],
        out_shape=[
            jax.ShapeDtypeStruct((B, CO, N), jnp.float32),
            jax.ShapeDtypeStruct((2, CO), jnp.float32),
        ],
    )(interp, uf, w0a, w0b)


def _mlp2_body(y1_ref, s1_ref, g_ref, be_ref, w1_ref, y2_ref, s2_ref):
    step = pl.program_id(0) * pl.num_programs(1) + pl.program_id(1)
    n = jnp.float32(B * N)
    mean = s1_ref[0, :] / n
    var = s1_ref[1, :] / n - mean * mean
    scale = g_ref[0] / jnp.sqrt(var + 1e-5)
    shift = be_ref[0] - mean * scale
    h = jnp.maximum(y1_ref[0] * scale[:, None] + shift[:, None], 0.0)
    y = lax.dot_general(w1_ref[...], h, (((1,), (0,)), ((), ())),
                        preferred_element_type=jnp.float32)
    y2_ref[0] = y
    st = jnp.concatenate([jnp.sum(y, axis=1)[None, :],
                          jnp.sum(y * y, axis=1)[None, :]], axis=0)

    @pl.when(step == 0)
    def _():
        s2_ref[...] = st

    @pl.when(step != 0)
    def _():
        s2_ref[...] += st


def _mlp2(y1, s1, g0, be0, w1):
    return pl.pallas_call(
        _mlp2_body,
        grid=(B, NT2),
        in_specs=[
            pl.BlockSpec((1, CO, TN2), lambda b, t: (b, 0, t)),
            pl.BlockSpec((2, CO), lambda b, t: (0, 0)),
            pl.BlockSpec((1, CO), lambda b, t: (0, 0)),
            pl.BlockSpec((1, CO), lambda b, t: (0, 0)),
            pl.BlockSpec((CO, CO), lambda b, t: (0, 0)),
        ],
        out_specs=[
            pl.BlockSpec((1, CO, TN2), lambda b, t: (b, 0, t)),
            pl.BlockSpec((2, CO), lambda b, t: (0, 0)),
        ],
        out_shape=[
            jax.ShapeDtypeStruct((B, CO, N), jnp.float32),
            jax.ShapeDtypeStruct((2, CO), jnp.float32),
        ],
    )(y1, s1, g0, be0, w1)


def _bnout_body(y2_ref, s2_ref, g_ref, be_ref, out_ref):
    n = jnp.float32(B * N)
    mean = s2_ref[0, :] / n
    var = s2_ref[1, :] / n - mean * mean
    scale = g_ref[0] / jnp.sqrt(var + 1e-5)
    shift = be_ref[0] - mean * scale
    out_ref[0] = jnp.maximum(y2_ref[0] * scale[:, None] + shift[:, None], 0.0)


def _bnout(y2, s2, g1, be1):
    return pl.pallas_call(
        _bnout_body,
        grid=(B, NT2),
        in_specs=[
            pl.BlockSpec((1, CO, TN2), lambda b, t: (b, 0, t)),
            pl.BlockSpec((2, CO), lambda b, t: (0, 0)),
            pl.BlockSpec((1, CO), lambda b, t: (0, 0)),
            pl.BlockSpec((1, CO), lambda b, t: (0, 0)),
        ],
        out_specs=pl.BlockSpec((1, CO, TN2), lambda b, t: (b, 0, t)),
        out_shape=jax.ShapeDtypeStruct((B, CO, N), jnp.float32),
    )(y2, s2, g1, be1)


def kernel(unknown, known, unknow_feats, known_feats, W0, g0, be0, W1, g1, be1):
    gidx, w48 = _knn(unknown, known)
    kft = jnp.transpose(known_feats, (0, 2, 1)).reshape(B * M, C2)
    interp = _sc_interp(gidx.reshape(B * N * K), w48.reshape(B * N * 48), kft)
    w0a = W0[:, :C2]
    w0b = W0[:, C2:]
    y1, s1 = _mlp1(interp, unknow_feats, w0a, w0b)
    y2, s2 = _mlp2(y1, s1, g0.reshape(1, CO), be0.reshape(1, CO), W1)
    return _bnout(y2, s2, g1.reshape(1, CO), be1.reshape(1, CO))
